# baseline (speedup 1.0000x reference)
import jax
import jax.numpy as jnp
from jax.experimental import pallas as pl

_TOKENS = 8192
_DIM = 2048
_EXPERTS = 64
_BT = 1024


def _gate_kernel(x_ref, wt_ref, out_ref):
    logits = jnp.dot(x_ref[...], wt_ref[...],
                     preferred_element_type=jnp.float32)
    m = jnp.max(logits, axis=1, keepdims=True)
    e = jnp.exp(logits - m)
    out_ref[...] = e / jnp.sum(e, axis=1, keepdims=True)


def kernel(inputs, wg):
    wt = wg.T
    return pl.pallas_call(
        _gate_kernel,
        grid=(_TOKENS // _BT,),
        in_specs=[
            pl.BlockSpec((_BT, _DIM), lambda i: (i, 0)),
            pl.BlockSpec((_DIM, _EXPERTS), lambda i: (0, 0)),
        ],
        out_specs=pl.BlockSpec((_BT, _EXPERTS), lambda i: (i, 0)),
        out_shape=jax.ShapeDtypeStruct((_TOKENS, _EXPERTS), jnp.float32),
    )(inputs, wt)


# speedup vs baseline: 1.0639x; 1.0639x over previous
import jax
import jax.numpy as jnp
from jax.experimental import pallas as pl

_TOKENS = 8192
_DIM = 2048
_EXPERTS = 64
_BT = 1024


def _gate_kernel(x_ref, w_ref, out_ref):
    logits = jax.lax.dot_general(
        x_ref[...], w_ref[...],
        dimension_numbers=(((1,), (1,)), ((), ())),
        preferred_element_type=jnp.float32)
    m = jnp.max(logits, axis=1, keepdims=True)
    e = jnp.exp(logits - m)
    out_ref[...] = e / jnp.sum(e, axis=1, keepdims=True)


def kernel(inputs, wg):
    return pl.pallas_call(
        _gate_kernel,
        grid=(_TOKENS // _BT,),
        in_specs=[
            pl.BlockSpec((_BT, _DIM), lambda i: (i, 0)),
            pl.BlockSpec((_EXPERTS, _DIM), lambda i: (0, 0)),
        ],
        out_specs=pl.BlockSpec((_BT, _EXPERTS), lambda i: (i, 0)),
        out_shape=jax.ShapeDtypeStruct((_TOKENS, _EXPERTS), jnp.float32),
    )(inputs, wg)
